# baseline (speedup 1.0000x reference)
import functools

import jax
import jax.numpy as jnp
from jax import lax
from jax.experimental import pallas as pl
from jax.experimental.pallas import tpu as pltpu
from jax.experimental.pallas import tpu_sc as plsc

N = 10000
E = 320000
H = 128
G = 128
NT = 1

NB = 5
BN = N // NB

NC = 2
NS = 16
EPW = E // (NC * NS)
CH = 96
EPWP = 10176
NCHUNK = EPWP // CH
PKC = 128
NA = N + (EPWP - EPW)
ZCH = 40
NSLICE = N // ZCH


def _sc_aggregate(y, src2d, dst3d):
  mesh = plsc.VectorSubcoreMesh(core_axis_name="c", subcore_axis_name="s")

  @functools.partial(
      pl.kernel,
      mesh=mesh,
      out_type=jax.ShapeDtypeStruct((NC, N, H), jnp.float32),
      scratch_types=[
          pltpu.VMEM((EPWP,), jnp.int32),
          pltpu.VMEM((NCHUNK, CH), jnp.int32),
          pltpu.VMEM((CH, H), jnp.float32),
          pltpu.VMEM((CH, H), jnp.float32),
          pltpu.VMEM_SHARED((NA, H), jnp.float32),
          pltpu.SemaphoreType.DMA,
          pltpu.SemaphoreType.DMA,
      ],
  )
  def agg_kernel(y_hbm, src_hbm, dst_hbm, out_hbm, sidx, didx, rows0, rows1,
                 acc, g0, g1):
    c = lax.axis_index("c")
    s = lax.axis_index("s")
    wid = c * NS + s
    rows = (rows0, rows1)
    gsem = (g0, g1)

    cp_s = pltpu.async_copy(src_hbm.at[wid], sidx, g0)
    cp_d = pltpu.async_copy(dst_hbm.at[wid], didx, g1)

    zv = jnp.zeros((16,), jnp.float32)

    @pl.loop(0, CH)
    def _(r):
      @pl.loop(0, H, step=16)
      def _(j):
        rows0[r, pl.ds(j, 16)] = zv

    @pl.loop(0, (NSLICE + NS - 1) // NS)
    def _(k):
      j = s + k * NS
      @pl.when(j < NSLICE)
      def _():
        pltpu.sync_copy(rows0.at[pl.ds(0, ZCH)], acc.at[pl.ds(j * ZCH, ZCH)])

    cp_s.wait()
    cp_d.wait()
    plsc.subcore_barrier()

    def gather(b, k):
      pltpu.async_copy(y_hbm.at[sidx.at[pl.ds(k * CH, CH)]], rows[b],
                       gsem[b])

    def gwait(b):
      pltpu.make_async_copy(y_hbm.at[sidx.at[pl.ds(0, CH)]], rows[b],
                            gsem[b]).wait()

    def scat(b, k):
      pltpu.sync_copy(rows[b], acc.at[didx.at[k]], add=True)

    gather(0, 0)

    @pl.loop(0, NCHUNK // 2)
    def _(kk):
      for b in (0, 1):
        k = 2 * kk + b
        nb = 1 - b

        @pl.when(k + 1 < NCHUNK)
        def _():
          gather(nb, k + 1)

        gwait(b)
        scat(b, k)

    plsc.subcore_barrier()

    pltpu.sync_copy(acc.at[pl.ds(s * 624, 624)],
                    out_hbm.at[c, pl.ds(s * 624, 624)])

    @pl.when(s == 0)
    def _():
      pltpu.sync_copy(acc.at[pl.ds(NS * 624, N - NS * 624)],
                      out_hbm.at[c, pl.ds(NS * 624, N - NS * 624)])

  return agg_kernel(y, src2d, dst3d)


def _tc_layer(parts, x, W, b, Wr, br):
  def body(p_ref, x_ref, w_ref, b_ref, wr_ref, br_ref, h_ref):
    agg = p_ref[0] + p_ref[1]
    hw = jnp.dot(agg, w_ref[...], preferred_element_type=jnp.float32)
    r = jnp.dot(x_ref[...], wr_ref[...], preferred_element_type=jnp.float32)
    h_ref[...] = (jnp.maximum(hw + b_ref[...], 0.0)
                  + jnp.maximum(r + br_ref[...], 0.0))

  return pl.pallas_call(
      body,
      grid=(NB,),
      in_specs=[
          pl.BlockSpec((NC, BN, H), lambda i: (0, i, 0)),
          pl.BlockSpec((BN, H), lambda i: (i, 0)),
          pl.BlockSpec((H, H), lambda i: (0, 0)),
          pl.BlockSpec((1, H), lambda i: (0, 0)),
          pl.BlockSpec((H, H), lambda i: (0, 0)),
          pl.BlockSpec((1, H), lambda i: (0, 0)),
      ],
      out_specs=pl.BlockSpec((BN, H), lambda i: (i, 0)),
      out_shape=jax.ShapeDtypeStruct((N, H), jnp.float32),
  )(parts, x, W, b.reshape(1, H), Wr, br.reshape(1, H))


def _tc_final(parts, h1, W2, b2, Wr2, br2, Wg, bg, ids, Wp1, bp1, gamma,
              beta, Wp2, bp2):
  def body(p_ref, h1_ref, w2_ref, b2_ref, wr2_ref, br2_ref, wg_ref, bg_ref,
           ids_ref, wp1_ref, bp1_ref, ga_ref, be_ref, wp2_ref, bp2_ref,
           out_ref, wsum, hmax):
    i = pl.program_id(0)

    @pl.when(i == 0)
    def _():
      wsum[...] = jnp.zeros((G, H), jnp.float32)
      hmax[...] = jnp.full((G, H), -jnp.inf, jnp.float32)

    agg = p_ref[0] + p_ref[1]
    hw = jnp.dot(agg, w2_ref[...], preferred_element_type=jnp.float32)
    r = jnp.dot(h1_ref[...], wr2_ref[...], preferred_element_type=jnp.float32)
    h = (jnp.maximum(hw + b2_ref[...], 0.0)
         + jnp.maximum(r + br2_ref[...], 0.0))
    gate = jax.nn.sigmoid(
        jnp.dot(h, wg_ref[...], preferred_element_type=jnp.float32)
        + bg_ref[0, 0])
    gh = gate * h
    ids = ids_ref[...]
    onehot = (ids == lax.broadcasted_iota(jnp.int32, (1, G), 1)
              ).astype(jnp.float32)
    wsum[...] += lax.dot_general(
        onehot, gh, (((0,), (0,)), ((), ())),
        precision=lax.Precision.HIGHEST,
        preferred_element_type=jnp.float32)

    lo = ids[0, 0]
    hi = ids[BN - 1, 0]

    def gbody(g, carry):
      m = jnp.where(ids == g, h, -jnp.inf)
      row = jnp.max(m, axis=0, keepdims=True)
      hmax[pl.ds(g, 1), :] = jnp.maximum(hmax[pl.ds(g, 1), :], row)
      return carry

    lax.fori_loop(lo, hi + 1, gbody, 0)

    @pl.when(i == NB - 1)
    def _():
      gf = jnp.concatenate([wsum[...], hmax[...]], axis=1)
      z = jnp.dot(gf, wp1_ref[...], preferred_element_type=jnp.float32)
      z = jnp.maximum(z + bp1_ref[...], 0.0)
      mu = jnp.mean(z, axis=0, keepdims=True)
      var = jnp.mean((z - mu) * (z - mu), axis=0, keepdims=True)
      zn = (z - mu) / jnp.sqrt(var + 1e-5) * ga_ref[...] + be_ref[...]
      out_ref[...] = (
          jnp.dot(zn, wp2_ref[...], preferred_element_type=jnp.float32)
          + bp2_ref[...])

  return pl.pallas_call(
      body,
      grid=(NB,),
      in_specs=[
          pl.BlockSpec((NC, BN, H), lambda i: (0, i, 0)),
          pl.BlockSpec((BN, H), lambda i: (i, 0)),
          pl.BlockSpec((H, H), lambda i: (0, 0)),
          pl.BlockSpec((1, H), lambda i: (0, 0)),
          pl.BlockSpec((H, H), lambda i: (0, 0)),
          pl.BlockSpec((1, H), lambda i: (0, 0)),
          pl.BlockSpec((H, NT), lambda i: (0, 0)),
          pl.BlockSpec((1, 1), lambda i: (0, 0)),
          pl.BlockSpec((BN, 1), lambda i: (i, 0)),
          pl.BlockSpec((2 * H, H), lambda i: (0, 0)),
          pl.BlockSpec((1, H), lambda i: (0, 0)),
          pl.BlockSpec((1, H), lambda i: (0, 0)),
          pl.BlockSpec((1, H), lambda i: (0, 0)),
          pl.BlockSpec((H, NT), lambda i: (0, 0)),
          pl.BlockSpec((1, NT), lambda i: (0, 0)),
      ],
      out_specs=pl.BlockSpec((G, NT), lambda i: (0, 0)),
      out_shape=jax.ShapeDtypeStruct((G, NT), jnp.float32),
      scratch_shapes=[
          pltpu.VMEM((G, H), jnp.float32),
          pltpu.VMEM((G, H), jnp.float32),
      ],
  )(parts, h1, W2, b2.reshape(1, H), Wr2, br2.reshape(1, H),
    Wg, bg.reshape(1, 1), ids.reshape(N, 1),
    Wp1, bp1.reshape(1, H), gamma.reshape(1, H), beta.reshape(1, H),
    Wp2, bp2.reshape(1, NT))


@jax.jit
def kernel(x, edge_index, graph_ids, W1, b1, Wr1, br1, W2, b2, Wr2, br2,
           Wg, bg, Wp1, bp1, gamma, beta, Wp2, bp2):
  pad = EPWP - EPW
  src2d = jnp.pad(edge_index[0].reshape(NC * NS, EPW), ((0, 0), (0, pad)),
                  constant_values=0)
  dpad = jnp.broadcast_to(N + jnp.arange(pad, dtype=jnp.int32),
                          (NC * NS, pad))
  dst3d = jnp.concatenate(
      [edge_index[1].reshape(NC * NS, EPW), dpad],
      axis=1).reshape(NC * NS, NCHUNK, CH)

  parts1 = _sc_aggregate(x, src2d, dst3d)
  h1 = _tc_layer(parts1, x, W1, b1, Wr1, br1)
  parts2 = _sc_aggregate(h1, src2d, dst3d)
  return _tc_final(parts2, h1, W2, b2, Wr2, br2, Wg, bg, graph_ids,
                   Wp1, bp1, gamma, beta, Wp2, bp2)


# speedup vs baseline: 1.6451x; 1.6451x over previous
import functools

import jax
import jax.numpy as jnp
from jax import lax
from jax.experimental import pallas as pl
from jax.experimental.pallas import tpu as pltpu
from jax.experimental.pallas import tpu_sc as plsc

N = 10000
E = 320000
H = 128
G = 128
NT = 1

NB = 5
BN = N // NB

NC = 2
NS = 16
EPW = E // (NC * NS)
CH = 125
EPWP = 10000
NCHUNK = EPWP // CH
PKC = 128
NA = N + (EPWP - EPW)
ZCH = 40
NSLICE = N // ZCH


def _sc_aggregate(y, src2d, dst3d):
  mesh = plsc.VectorSubcoreMesh(core_axis_name="c", subcore_axis_name="s")

  @functools.partial(
      pl.kernel,
      mesh=mesh,
      out_type=jax.ShapeDtypeStruct((NC, N, H), jnp.float32),
      scratch_types=[
          pltpu.VMEM((NCHUNK, CH), jnp.int32),
          pltpu.VMEM((NCHUNK, CH), jnp.int32),
          pltpu.VMEM((CH, H), jnp.float32),
          pltpu.VMEM_SHARED((NA, H), jnp.float32),
          pltpu.SemaphoreType.DMA,
          pltpu.SemaphoreType.DMA,
      ],
  )
  def agg_kernel(y_hbm, src_hbm, dst_hbm, out_hbm, sidx, didx, rows0,
                 acc, g0, g1):
    c = lax.axis_index("c")
    s = lax.axis_index("s")
    wid = c * NS + s

    cp_s = pltpu.async_copy(src_hbm.at[wid], sidx, g0)
    cp_d = pltpu.async_copy(dst_hbm.at[wid], didx, g1)

    zv = jnp.zeros((16,), jnp.float32)

    @pl.loop(0, CH)
    def _(r):
      @pl.loop(0, H, step=16)
      def _(j):
        rows0[r, pl.ds(j, 16)] = zv

    @pl.loop(0, (NSLICE + NS - 1) // NS)
    def _(k):
      j = s + k * NS
      @pl.when(j < NSLICE)
      def _():
        pltpu.sync_copy(rows0.at[pl.ds(0, ZCH)], acc.at[pl.ds(j * ZCH, ZCH)])

    cp_s.wait()
    cp_d.wait()
    plsc.subcore_barrier()

    @pl.loop(0, NCHUNK)
    def _(k):
      pltpu.async_copy(y_hbm.at[sidx.at[k]], rows0, g0).wait()
      pltpu.sync_copy(rows0, acc.at[didx.at[k]], add=True)

    plsc.subcore_barrier()

    pltpu.sync_copy(acc.at[pl.ds(s * 624, 624)],
                    out_hbm.at[c, pl.ds(s * 624, 624)])

    @pl.when(s == 0)
    def _():
      pltpu.sync_copy(acc.at[pl.ds(NS * 624, N - NS * 624)],
                      out_hbm.at[c, pl.ds(NS * 624, N - NS * 624)])

  return agg_kernel(y, src2d, dst3d)


def _tc_layer(parts, x, W, b, Wr, br):
  def body(p_ref, x_ref, w_ref, b_ref, wr_ref, br_ref, h_ref):
    agg = p_ref[0] + p_ref[1]
    hw = jnp.dot(agg, w_ref[...], preferred_element_type=jnp.float32)
    r = jnp.dot(x_ref[...], wr_ref[...], preferred_element_type=jnp.float32)
    h_ref[...] = (jnp.maximum(hw + b_ref[...], 0.0)
                  + jnp.maximum(r + br_ref[...], 0.0))

  return pl.pallas_call(
      body,
      grid=(NB,),
      in_specs=[
          pl.BlockSpec((NC, BN, H), lambda i: (0, i, 0)),
          pl.BlockSpec((BN, H), lambda i: (i, 0)),
          pl.BlockSpec((H, H), lambda i: (0, 0)),
          pl.BlockSpec((1, H), lambda i: (0, 0)),
          pl.BlockSpec((H, H), lambda i: (0, 0)),
          pl.BlockSpec((1, H), lambda i: (0, 0)),
      ],
      out_specs=pl.BlockSpec((BN, H), lambda i: (i, 0)),
      out_shape=jax.ShapeDtypeStruct((N, H), jnp.float32),
  )(parts, x, W, b.reshape(1, H), Wr, br.reshape(1, H))


def _tc_final(parts, h1, W2, b2, Wr2, br2, Wg, bg, ids, Wp1, bp1, gamma,
              beta, Wp2, bp2):
  def body(p_ref, h1_ref, w2_ref, b2_ref, wr2_ref, br2_ref, wg_ref, bg_ref,
           ids_ref, wp1_ref, bp1_ref, ga_ref, be_ref, wp2_ref, bp2_ref,
           out_ref, wsum, hmax):
    i = pl.program_id(0)

    @pl.when(i == 0)
    def _():
      wsum[...] = jnp.zeros((G, H), jnp.float32)
      hmax[...] = jnp.full((G, H), -jnp.inf, jnp.float32)

    agg = p_ref[0] + p_ref[1]
    hw = jnp.dot(agg, w2_ref[...], preferred_element_type=jnp.float32)
    r = jnp.dot(h1_ref[...], wr2_ref[...], preferred_element_type=jnp.float32)
    h = (jnp.maximum(hw + b2_ref[...], 0.0)
         + jnp.maximum(r + br2_ref[...], 0.0))
    gate = jax.nn.sigmoid(
        jnp.dot(h, wg_ref[...], preferred_element_type=jnp.float32)
        + bg_ref[0, 0])
    gh = gate * h
    ids = ids_ref[...]
    onehot = (ids == lax.broadcasted_iota(jnp.int32, (1, G), 1)
              ).astype(jnp.float32)
    wsum[...] += lax.dot_general(
        onehot, gh, (((0,), (0,)), ((), ())),
        precision=lax.Precision.HIGHEST,
        preferred_element_type=jnp.float32)

    lo = ids[0, 0]
    hi = ids[BN - 1, 0]

    def gbody(g, carry):
      m = jnp.where(ids == g, h, -jnp.inf)
      row = jnp.max(m, axis=0, keepdims=True)
      hmax[pl.ds(g, 1), :] = jnp.maximum(hmax[pl.ds(g, 1), :], row)
      return carry

    lax.fori_loop(lo, hi + 1, gbody, 0)

    @pl.when(i == NB - 1)
    def _():
      gf = jnp.concatenate([wsum[...], hmax[...]], axis=1)
      z = jnp.dot(gf, wp1_ref[...], preferred_element_type=jnp.float32)
      z = jnp.maximum(z + bp1_ref[...], 0.0)
      mu = jnp.mean(z, axis=0, keepdims=True)
      var = jnp.mean((z - mu) * (z - mu), axis=0, keepdims=True)
      zn = (z - mu) / jnp.sqrt(var + 1e-5) * ga_ref[...] + be_ref[...]
      out_ref[...] = (
          jnp.dot(zn, wp2_ref[...], preferred_element_type=jnp.float32)
          + bp2_ref[...])

  return pl.pallas_call(
      body,
      grid=(NB,),
      in_specs=[
          pl.BlockSpec((NC, BN, H), lambda i: (0, i, 0)),
          pl.BlockSpec((BN, H), lambda i: (i, 0)),
          pl.BlockSpec((H, H), lambda i: (0, 0)),
          pl.BlockSpec((1, H), lambda i: (0, 0)),
          pl.BlockSpec((H, H), lambda i: (0, 0)),
          pl.BlockSpec((1, H), lambda i: (0, 0)),
          pl.BlockSpec((H, NT), lambda i: (0, 0)),
          pl.BlockSpec((1, 1), lambda i: (0, 0)),
          pl.BlockSpec((BN, 1), lambda i: (i, 0)),
          pl.BlockSpec((2 * H, H), lambda i: (0, 0)),
          pl.BlockSpec((1, H), lambda i: (0, 0)),
          pl.BlockSpec((1, H), lambda i: (0, 0)),
          pl.BlockSpec((1, H), lambda i: (0, 0)),
          pl.BlockSpec((H, NT), lambda i: (0, 0)),
          pl.BlockSpec((1, NT), lambda i: (0, 0)),
      ],
      out_specs=pl.BlockSpec((G, NT), lambda i: (0, 0)),
      out_shape=jax.ShapeDtypeStruct((G, NT), jnp.float32),
      scratch_shapes=[
          pltpu.VMEM((G, H), jnp.float32),
          pltpu.VMEM((G, H), jnp.float32),
      ],
  )(parts, h1, W2, b2.reshape(1, H), Wr2, br2.reshape(1, H),
    Wg, bg.reshape(1, 1), ids.reshape(N, 1),
    Wp1, bp1.reshape(1, H), gamma.reshape(1, H), beta.reshape(1, H),
    Wp2, bp2.reshape(1, NT))


@jax.jit
def kernel(x, edge_index, graph_ids, W1, b1, Wr1, br1, W2, b2, Wr2, br2,
           Wg, bg, Wp1, bp1, gamma, beta, Wp2, bp2):
  pad = EPWP - EPW
  src2d = jnp.pad(edge_index[0].reshape(NC * NS, EPW), ((0, 0), (0, pad)),
                  constant_values=0).reshape(NC * NS, NCHUNK, CH)
  dpad = jnp.broadcast_to(N + jnp.arange(pad, dtype=jnp.int32),
                          (NC * NS, pad))
  dst3d = jnp.concatenate(
      [edge_index[1].reshape(NC * NS, EPW), dpad],
      axis=1).reshape(NC * NS, NCHUNK, CH)

  parts1 = _sc_aggregate(x, src2d, dst3d)
  h1 = _tc_layer(parts1, x, W1, b1, Wr1, br1)
  parts2 = _sc_aggregate(h1, src2d, dst3d)
  return _tc_final(parts2, h1, W2, b2, Wr2, br2, Wg, bg, graph_ids,
                   Wp1, bp1, gamma, beta, Wp2, bp2)
